# baseline (speedup 1.0000x reference)
import jax
import jax.numpy as jnp
from jax import lax
from jax.experimental import pallas as pl
from jax.experimental.pallas import tpu as pltpu

N_DOMAIN = 8
EPS = 1e-05
ROWS = 16384
COLS = 1024
BR = 2048
NB = ROWS // BR
CQ = 256
NQ = COLS // CQ


def _onehot_t(y_ref):
    yv = y_ref[0]
    ids = lax.broadcasted_iota(jnp.int32, (N_DOMAIN, BR), 0)
    return (ids == yv).astype(jnp.float32)


def _copy(x_any, buf, sems, q, blk, slot):
    return pltpu.make_async_copy(
        x_any.at[pl.ds(blk * BR, BR), pl.ds(q * CQ, CQ)],
        buf.at[pl.ds(blk * BR, BR), :], sems.at[slot, blk])


def _pipe_kernel(y_ref, g_ref, b_ref, x_any, out_ref,
                 xbuf0, xbuf1, sums, sumsq, cnt, atab, btab, sems):
    s = pl.program_id(0)
    i = pl.program_id(1)
    par = s % 2

    def with_buf(fn):
        @pl.when(par == 0)
        def _b0():
            fn(xbuf0)

        @pl.when(par == 1)
        def _b1():
            fn(xbuf1)

    oh = _onehot_t(y_ref)

    @pl.when(s < NQ)
    def _phase0():
        @pl.when(jnp.logical_and(s == 0, i == 0))
        def _very_first():
            def start0(buf):
                _copy(x_any, buf, sems, 0, 0, 0).start()
            with_buf(start0)

        @pl.when(i == 0)
        def _zero():
            sums[par] = jnp.zeros_like(sums[par])
            sumsq[par] = jnp.zeros_like(sumsq[par])

        @pl.when(jnp.logical_and(s == 0, i == 0))
        def _zero_cnt():
            cnt[...] = jnp.zeros_like(cnt)

        @pl.when(i + 1 < NB)
        def _next():
            def startn(buf):
                _copy(x_any, buf, sems, s, i + 1, par).start()
            with_buf(startn)

        @pl.when(jnp.logical_and(i == NB - 1, s + 1 < NQ))
        def _nextq():
            @pl.when(par == 0)
            def _q1():
                _copy(x_any, xbuf1, sems, s + 1, 0, 1 - par).start()

            @pl.when(par == 1)
            def _q0():
                _copy(x_any, xbuf0, sems, s + 1, 0, 1 - par).start()

        def accum(buf):
            _copy(x_any, buf, sems, s, i, par).wait()
            xb = buf[pl.ds(i * BR, BR), :]
            sums[par] += lax.dot_general(
                oh, xb, (((1,), (0,)), ((), ())),
                preferred_element_type=jnp.float32)
            sumsq[par] += lax.dot_general(
                oh, xb * xb, (((1,), (0,)), ((), ())),
                preferred_element_type=jnp.float32)
        with_buf(accum)

        @pl.when(s == 0)
        def _count():
            cnt[...] += jnp.broadcast_to(
                jnp.sum(oh, axis=1, keepdims=True), cnt.shape)

    @pl.when(s >= 1)
    def _phase1():
        qpar = 1 - par

        @pl.when(i == 0)
        def _tables():
            c = cnt[:, :1]
            denom = jnp.maximum(c, 1.0)
            mean = sums[qpar] / denom
            var = jnp.maximum(sumsq[qpar] / denom - mean * mean, 0.0)
            scale = g_ref[...] * lax.rsqrt(var + EPS)
            multi = c > 1.0
            atab[qpar] = jnp.where(multi, scale, 1.0)
            btab[qpar] = jnp.where(multi, b_ref[...] - mean * scale, 0.0)

        row_a = lax.dot_general(oh, atab[qpar], (((0,), (0,)), ((), ())),
                                preferred_element_type=jnp.float32)
        row_b = lax.dot_general(oh, btab[qpar], (((0,), (0,)), ((), ())),
                                preferred_element_type=jnp.float32)

        @pl.when(qpar == 0)
        def _a0():
            out_ref[...] = xbuf0[pl.ds(i * BR, BR), :] * row_a + row_b

        @pl.when(qpar == 1)
        def _a1():
            out_ref[...] = xbuf1[pl.ds(i * BR, BR), :] * row_a + row_b


@jax.jit
def kernel(x, y, gamma, beta):
    y3 = y.astype(jnp.int32).reshape(NB, 1, BR)
    out = pl.pallas_call(
        _pipe_kernel,
        grid=(NQ + 1, NB),
        in_specs=[
            pl.BlockSpec((1, 1, BR), lambda s, i: (i, 0, 0)),
            pl.BlockSpec((1, CQ), lambda s, i: (0, jnp.maximum(s - 1, 0))),
            pl.BlockSpec((1, CQ), lambda s, i: (0, jnp.maximum(s - 1, 0))),
            pl.BlockSpec(memory_space=pl.ANY),
        ],
        out_specs=pl.BlockSpec(
            (BR, CQ),
            lambda s, i: (jnp.where(s > 0, i, 0), jnp.maximum(s - 1, 0))),
        out_shape=jax.ShapeDtypeStruct((ROWS, COLS), jnp.float32),
        scratch_shapes=[
            pltpu.VMEM((ROWS, CQ), jnp.float32),
            pltpu.VMEM((ROWS, CQ), jnp.float32),
            pltpu.VMEM((2, N_DOMAIN, CQ), jnp.float32),
            pltpu.VMEM((2, N_DOMAIN, CQ), jnp.float32),
            pltpu.VMEM((N_DOMAIN, 128), jnp.float32),
            pltpu.VMEM((2, N_DOMAIN, CQ), jnp.float32),
            pltpu.VMEM((2, N_DOMAIN, CQ), jnp.float32),
            pltpu.SemaphoreType.DMA((2, NB)),
        ],
    )(y3, gamma, beta, x)
    return out


# speedup vs baseline: 1.3293x; 1.3293x over previous
import jax
import jax.numpy as jnp
from jax import lax
from jax.experimental import pallas as pl
from jax.experimental.pallas import tpu as pltpu

N_DOMAIN = 8
EPS = 1e-05
ROWS = 16384
COLS = 1024
BR = 4096
NB = ROWS // BR
COLH = 512
NH = COLS // COLH


def _onehot_t(y_ref):
    yv = y_ref[0]
    ids = lax.broadcasted_iota(jnp.int32, (N_DOMAIN, BR), 0)
    return (ids == yv).astype(jnp.float32)


def _fused_kernel(y_ref, g_ref, b_ref, x_any, out_ref,
                  xbuf, sums, sumsq, cnt, atab, btab, sems):
    h = pl.program_id(0)
    p = pl.program_id(1)
    i = pl.program_id(2)

    @pl.when(p == 0)
    def _phase0():
        @pl.when(jnp.logical_and(h == 0, i == 0))
        def _first():
            pltpu.make_async_copy(
                x_any.at[pl.ds(0, BR), pl.ds(0, COLH)],
                xbuf.at[pl.ds(0, BR), :], sems.at[0]).start()

        @pl.when(i == 0)
        def _zero():
            sums[...] = jnp.zeros_like(sums)
            sumsq[...] = jnp.zeros_like(sumsq)
            cnt[...] = jnp.zeros_like(cnt)

        @pl.when(i + 1 < NB)
        def _next():
            pltpu.make_async_copy(
                x_any.at[pl.ds((i + 1) * BR, BR), pl.ds(h * COLH, COLH)],
                xbuf.at[pl.ds((i + 1) * BR, BR), :], sems.at[i + 1]).start()

        pltpu.make_async_copy(
            x_any.at[pl.ds(i * BR, BR), pl.ds(h * COLH, COLH)],
            xbuf.at[pl.ds(i * BR, BR), :], sems.at[i]).wait()

        xb = xbuf[pl.ds(i * BR, BR), :]
        oh = _onehot_t(y_ref)
        sums[...] += lax.dot_general(
            oh, xb, (((1,), (0,)), ((), ())),
            preferred_element_type=jnp.float32)
        sumsq[...] += lax.dot_general(
            oh, xb * xb, (((1,), (0,)), ((), ())),
            preferred_element_type=jnp.float32)
        cnt[...] += jnp.broadcast_to(
            jnp.sum(oh, axis=1, keepdims=True), cnt.shape)

    @pl.when(p == 1)
    def _phase1():
        @pl.when(i == 0)
        def _tables():
            c = cnt[:, :1]
            denom = jnp.maximum(c, 1.0)
            mean = sums[...] / denom
            var = jnp.maximum(sumsq[...] / denom - mean * mean, 0.0)
            scale = g_ref[...] * lax.rsqrt(var + EPS)
            multi = c > 1.0
            atab[...] = jnp.where(multi, scale, 1.0)
            btab[...] = jnp.where(multi, b_ref[...] - mean * scale, 0.0)

        oh = _onehot_t(y_ref)
        row_a = lax.dot_general(oh, atab[...], (((0,), (0,)), ((), ())),
                                preferred_element_type=jnp.float32)
        row_b = lax.dot_general(oh, btab[...], (((0,), (0,)), ((), ())),
                                preferred_element_type=jnp.float32)
        out_ref[...] = xbuf[pl.ds(i * BR, BR), :] * row_a + row_b

        @pl.when(jnp.logical_and(i == NB - 1, h + 1 < NH))
        def _prefetch_next_half():
            pltpu.make_async_copy(
                x_any.at[pl.ds(0, BR), pl.ds((h + 1) * COLH, COLH)],
                xbuf.at[pl.ds(0, BR), :], sems.at[0]).start()


@jax.jit
def kernel(x, y, gamma, beta):
    y3 = y.astype(jnp.int32).reshape(NB, 1, BR)
    out = pl.pallas_call(
        _fused_kernel,
        grid=(NH, 2, NB),
        in_specs=[
            pl.BlockSpec((1, 1, BR), lambda h, p, i: (i, 0, 0)),
            pl.BlockSpec((1, COLH), lambda h, p, i: (0, h)),
            pl.BlockSpec((1, COLH), lambda h, p, i: (0, h)),
            pl.BlockSpec(memory_space=pl.ANY),
        ],
        out_specs=pl.BlockSpec((BR, COLH), lambda h, p, i: (i * p, h)),
        out_shape=jax.ShapeDtypeStruct((ROWS, COLS), jnp.float32),
        scratch_shapes=[
            pltpu.VMEM((ROWS, COLH), jnp.float32),
            pltpu.VMEM((N_DOMAIN, COLH), jnp.float32),
            pltpu.VMEM((N_DOMAIN, COLH), jnp.float32),
            pltpu.VMEM((N_DOMAIN, 128), jnp.float32),
            pltpu.VMEM((N_DOMAIN, COLH), jnp.float32),
            pltpu.VMEM((N_DOMAIN, COLH), jnp.float32),
            pltpu.SemaphoreType.DMA((NB,)),
        ],
    )(y3, gamma, beta, x)
    return out
